# baseline (speedup 1.0000x reference)
import functools

import jax
import jax.numpy as jnp
from jax import lax
from jax.experimental import pallas as pl
from jax.experimental.pallas import tpu as pltpu
from jax.experimental.pallas import tpu_sc as plsc

NC = 2
NS = 16
NW = NC * NS
L = 16

D = 128
S = 10

CH = 40
ROWS = CH * S
GSUB = 100
NG = ROWS // GSUB


def _sc_neighbor_sum(cnt_hi, cnt_lo, w_hi,
                     feat_hbm, idx_hbm, out_hbm,
                     idx0, idx1, rows0, rows1, acc0, acc1,
                     isem0, isem1, gsem0, gsem1, osem0, osem1):
  cid = lax.axis_index("c")
  sid = lax.axis_index("s")
  wid = sid * NC + cid
  cnt = jnp.where(wid < w_hi, cnt_hi, cnt_lo)
  start = jnp.where(wid < w_hi, wid * cnt_hi,
                    w_hi * cnt_hi + (wid - w_hi) * cnt_lo)

  def pre_idx(ci, idxv, isem):
    pltpu.async_copy(idx_hbm.at[pl.ds((start + ci) * NG, NG)], idxv, isem)

  def wait_idx(idxv, isem):
    pltpu.make_async_copy(idx_hbm.at[pl.ds(0, NG)], idxv, isem).wait()

  def fire(idxv, rowsv, sem):
    for k in range(NG):
      pltpu.async_copy(feat_hbm.at[idxv.at[k]],
                       rowsv.at[pl.ds(k * GSUB, GSUB)], sem)

  def wait_rows(rowsv, sem):
    pltpu.make_async_copy(feat_hbm.at[pl.ds(0, ROWS)], rowsv, sem).wait()

  def wait_out(accv, sem):
    pltpu.make_async_copy(accv, out_hbm.at[pl.ds(0, CH)], sem).wait()

  def reduce_chunk(rowsv, accv):
    def node_body(n, c2):
      rbase = n * S
      for c in range(D // L):
        sl = pl.ds(c * L, L)
        v = [rowsv[rbase + r, sl] for r in range(S)]
        s01, s23 = v[0] + v[1], v[2] + v[3]
        s45, s67 = v[4] + v[5], v[6] + v[7]
        s89 = v[8] + v[9]
        accv[n, sl] = ((s01 + s23) + (s45 + s67) + s89) * 0.1
      return c2
    lax.fori_loop(0, CH, node_body, 0)

  pre_idx(0, idx0, isem0)
  pre_idx(1, idx1, isem1)
  wait_idx(idx0, isem0)
  fire(idx0, rows0, gsem0)
  wait_idx(idx1, isem1)
  fire(idx1, rows1, gsem1)

  def pair_body(jj, carry):
    j = jj * 2

    wait_rows(rows0, gsem0)
    @pl.when(j + 2 < cnt)
    def _():
      pre_idx(j + 2, idx0, isem0)
    @pl.when(jj > 0)
    def _():
      wait_out(acc0, osem0)
    reduce_chunk(rows0, acc0)
    pltpu.async_copy(acc0, out_hbm.at[pl.ds((start + j) * CH, CH)], osem0)
    @pl.when(j + 2 < cnt)
    def _():
      wait_idx(idx0, isem0)
      fire(idx0, rows0, gsem0)

    wait_rows(rows1, gsem1)
    @pl.when(j + 3 < cnt)
    def _():
      pre_idx(j + 3, idx1, isem1)
    @pl.when(jj > 0)
    def _():
      wait_out(acc1, osem1)
    reduce_chunk(rows1, acc1)
    pltpu.async_copy(acc1, out_hbm.at[pl.ds((start + j + 1) * CH, CH)],
                     osem1)
    @pl.when(j + 3 < cnt)
    def _():
      wait_idx(idx1, isem1)
      fire(idx1, rows1, gsem1)
    return carry

  lax.fori_loop(0, cnt // 2, pair_body, 0)
  wait_out(acc0, osem0)
  wait_out(acc1, osem1)


def _make_sc_kernel(b, cnt_hi, cnt_lo, w_hi):
  mesh = plsc.VectorSubcoreMesh(core_axis_name="c", subcore_axis_name="s",
                                num_cores=NC, num_subcores=NS)
  return pl.kernel(
      functools.partial(_sc_neighbor_sum, cnt_hi, cnt_lo, w_hi),
      out_type=jax.ShapeDtypeStruct((b, D), jnp.float32),
      mesh=mesh,
      scratch_types=[
          pltpu.VMEM((NG, GSUB), jnp.int32),
          pltpu.VMEM((NG, GSUB), jnp.int32),
          pltpu.VMEM((ROWS, D), jnp.float32),
          pltpu.VMEM((ROWS, D), jnp.float32),
          pltpu.VMEM((CH, D), jnp.float32),
          pltpu.VMEM((CH, D), jnp.float32),
          pltpu.SemaphoreType.DMA,
          pltpu.SemaphoreType.DMA,
          pltpu.SemaphoreType.DMA,
          pltpu.SemaphoreType.DMA,
          pltpu.SemaphoreType.DMA,
          pltpu.SemaphoreType.DMA,
      ],
  )


def _tc_rel_body(rel_ref, table_ref, out_ref):
  bn = rel_ref.shape[0]
  rel = rel_ref[...]
  iota = lax.broadcasted_iota(jnp.int32, (bn, D), 1)
  counts = jnp.zeros((bn, D), jnp.float32)
  for s in range(S):
    counts = counts + jnp.where(rel[:, s][:, None] == iota, 0.1, 0.0)
  out_ref[...] = jnp.dot(counts, table_ref[...],
                         preferred_element_type=jnp.float32)


def _tc_rel(rel, table_pad, bn=1000):
  b = rel.shape[0]
  return pl.pallas_call(
      _tc_rel_body,
      grid=(b // bn,),
      in_specs=[
          pl.BlockSpec((bn, S), lambda i: (i, 0)),
          pl.BlockSpec((D, D), lambda i: (0, 0)),
      ],
      out_specs=pl.BlockSpec((bn, D), lambda i: (i, 0)),
      out_shape=jax.ShapeDtypeStruct((b, D), jnp.float32),
  )(rel, table_pad)


def _tc_final_body(nbr_ref, relout_ref, w_ref, out_ref):
  out = lax.dot_general(nbr_ref[...], w_ref[...], (((1,), (1,)), ((), ())),
                        preferred_element_type=jnp.float32)
  out_ref[...] = jnp.maximum(out + relout_ref[...], 0.0)


def _tc_final(nbr_sum, rel_out, weight, bn=1000):
  b = nbr_sum.shape[0]
  return pl.pallas_call(
      _tc_final_body,
      grid=(b // bn,),
      in_specs=[
          pl.BlockSpec((bn, D), lambda i: (i, 0)),
          pl.BlockSpec((bn, D), lambda i: (i, 0)),
          pl.BlockSpec((D, D), lambda i: (0, 0)),
      ],
      out_specs=pl.BlockSpec((bn, D), lambda i: (i, 0)),
      out_shape=jax.ShapeDtypeStruct((b, D), jnp.float32),
  )(nbr_sum, rel_out, weight)


def _nbr_sum_half(neighbors, node_features):
  b = neighbors.shape[0]
  total_chunks = (b * S) // (NG * GSUB)
  assert total_chunks % 2 == 0
  npairs = total_chunks // 2
  base = npairs // NW
  w_hi = npairs - base * NW
  cnt_hi, cnt_lo = 2 * (base + 1), 2 * base
  if w_hi == 0:
    w_hi, cnt_hi = NW, cnt_lo
  idx = neighbors.reshape((b * S) // GSUB, GSUB)
  return _make_sc_kernel(b, cnt_hi, cnt_lo, w_hi)(node_features, idx)


def kernel(nodes, sampled_neighbors, sampled_relations, node_features, weight,
           relation_table):
  del nodes
  b, s = sampled_neighbors.shape
  assert s == S and node_features.shape[1] == D

  b1 = ((b // 2 + 1999) // 2000) * 2000
  assert (b1 * S) % (2 * NG * GSUB) == 0 and ((b - b1) * S) % (
      2 * NG * GSUB) == 0 and b % 2000 == 0

  table_pad = jnp.pad(relation_table,
                      ((0, D - relation_table.shape[0]), (0, 0)))

  nbr1 = _nbr_sum_half(sampled_neighbors[:b1], node_features)
  nbr2 = _nbr_sum_half(sampled_neighbors[b1:], node_features)
  rel1 = _tc_rel(sampled_relations[:b1], table_pad)
  rel2 = _tc_rel(sampled_relations[b1:], table_pad)
  out1 = _tc_final(nbr1, rel1, weight)
  out2 = _tc_final(nbr2, rel2, weight)
  return jnp.concatenate([out1, out2], axis=0)


# speedup vs baseline: 1.0182x; 1.0182x over previous
import functools

import jax
import jax.numpy as jnp
from jax import lax
from jax.experimental import pallas as pl
from jax.experimental.pallas import tpu as pltpu
from jax.experimental.pallas import tpu_sc as plsc

NC = 2
NS = 16
NW = NC * NS
L = 16

D = 128
S = 10

CH = 40
ROWS = CH * S
GSUB = 100
NG = ROWS // GSUB


def _sc_neighbor_sum(cnt_hi, cnt_lo, w_hi,
                     feat_hbm, idx_hbm, out_hbm,
                     idx0, idx1, rows0, rows1, acc0, acc1,
                     isem0, isem1, gsem0, gsem1, osem0, osem1):
  cid = lax.axis_index("c")
  sid = lax.axis_index("s")
  wid = sid * NC + cid
  cnt = jnp.where(wid < w_hi, cnt_hi, cnt_lo)
  start = jnp.where(wid < w_hi, wid * cnt_hi,
                    w_hi * cnt_hi + (wid - w_hi) * cnt_lo)

  def pre_idx(ci, idxv, isem):
    pltpu.async_copy(idx_hbm.at[pl.ds((start + ci) * NG, NG)], idxv, isem)

  def wait_idx(idxv, isem):
    pltpu.make_async_copy(idx_hbm.at[pl.ds(0, NG)], idxv, isem).wait()

  def fire(idxv, rowsv, sem):
    for k in range(NG):
      pltpu.async_copy(feat_hbm.at[idxv.at[k]],
                       rowsv.at[pl.ds(k * GSUB, GSUB)], sem)

  def wait_rows(rowsv, sem):
    pltpu.make_async_copy(feat_hbm.at[pl.ds(0, ROWS)], rowsv, sem).wait()

  def wait_out(accv, sem):
    pltpu.make_async_copy(accv, out_hbm.at[pl.ds(0, CH)], sem).wait()

  def reduce_chunk(rowsv, accv):
    def node_body(n, c2):
      rbase = n * S
      for c in range(D // L):
        sl = pl.ds(c * L, L)
        v = [rowsv[rbase + r, sl] for r in range(S)]
        s01, s23 = v[0] + v[1], v[2] + v[3]
        s45, s67 = v[4] + v[5], v[6] + v[7]
        s89 = v[8] + v[9]
        accv[n, sl] = ((s01 + s23) + (s45 + s67) + s89) * 0.1
      return c2
    lax.fori_loop(0, CH, node_body, 0)

  pre_idx(0, idx0, isem0)
  pre_idx(1, idx1, isem1)
  wait_idx(idx0, isem0)
  fire(idx0, rows0, gsem0)
  wait_idx(idx1, isem1)
  fire(idx1, rows1, gsem1)

  def pair_body(jj, carry):
    j = jj * 2

    wait_rows(rows0, gsem0)
    @pl.when(j + 2 < cnt)
    def _():
      pre_idx(j + 2, idx0, isem0)
    @pl.when(jj > 0)
    def _():
      wait_out(acc0, osem0)
    reduce_chunk(rows0, acc0)
    pltpu.async_copy(acc0, out_hbm.at[pl.ds((start + j) * CH, CH)], osem0)
    @pl.when(j + 2 < cnt)
    def _():
      wait_idx(idx0, isem0)
      fire(idx0, rows0, gsem0)

    wait_rows(rows1, gsem1)
    @pl.when(j + 3 < cnt)
    def _():
      pre_idx(j + 3, idx1, isem1)
    @pl.when(jj > 0)
    def _():
      wait_out(acc1, osem1)
    reduce_chunk(rows1, acc1)
    pltpu.async_copy(acc1, out_hbm.at[pl.ds((start + j + 1) * CH, CH)],
                     osem1)
    @pl.when(j + 3 < cnt)
    def _():
      wait_idx(idx1, isem1)
      fire(idx1, rows1, gsem1)
    return carry

  lax.fori_loop(0, cnt // 2, pair_body, 0)
  wait_out(acc0, osem0)
  wait_out(acc1, osem1)


def _make_sc_kernel(b, cnt_hi, cnt_lo, w_hi):
  mesh = plsc.VectorSubcoreMesh(core_axis_name="c", subcore_axis_name="s",
                                num_cores=NC, num_subcores=NS)
  return pl.kernel(
      functools.partial(_sc_neighbor_sum, cnt_hi, cnt_lo, w_hi),
      out_type=jax.ShapeDtypeStruct((b, D), jnp.float32),
      mesh=mesh,
      scratch_types=[
          pltpu.VMEM((NG, GSUB), jnp.int32),
          pltpu.VMEM((NG, GSUB), jnp.int32),
          pltpu.VMEM((ROWS, D), jnp.float32),
          pltpu.VMEM((ROWS, D), jnp.float32),
          pltpu.VMEM((CH, D), jnp.float32),
          pltpu.VMEM((CH, D), jnp.float32),
          pltpu.SemaphoreType.DMA,
          pltpu.SemaphoreType.DMA,
          pltpu.SemaphoreType.DMA,
          pltpu.SemaphoreType.DMA,
          pltpu.SemaphoreType.DMA,
          pltpu.SemaphoreType.DMA,
      ],
  )


def _tc_rel_body(rel_ref, table_ref, out_ref):
  bn = rel_ref.shape[0]
  rel = rel_ref[...]
  iota = lax.broadcasted_iota(jnp.int32, (bn, D), 1)
  counts = jnp.zeros((bn, D), jnp.float32)
  for s in range(S):
    counts = counts + jnp.where(rel[:, s][:, None] == iota, 0.1, 0.0)
  out_ref[...] = jnp.dot(counts, table_ref[...],
                         preferred_element_type=jnp.float32)


def _tc_rel(rel, table_pad, bn=1000):
  b = rel.shape[0]
  return pl.pallas_call(
      _tc_rel_body,
      grid=(b // bn,),
      in_specs=[
          pl.BlockSpec((bn, S), lambda i: (i, 0)),
          pl.BlockSpec((D, D), lambda i: (0, 0)),
      ],
      out_specs=pl.BlockSpec((bn, D), lambda i: (i, 0)),
      out_shape=jax.ShapeDtypeStruct((b, D), jnp.float32),
  )(rel, table_pad)


def _tc_final_body(nbr_ref, relout_ref, w_ref, out_ref):
  out = lax.dot_general(nbr_ref[...], w_ref[...], (((1,), (1,)), ((), ())),
                        preferred_element_type=jnp.float32)
  out_ref[...] = jnp.maximum(out + relout_ref[...], 0.0)


def _tc_final(nbr_sum, rel_out, weight, bn=1000):
  b = nbr_sum.shape[0]
  return pl.pallas_call(
      _tc_final_body,
      grid=(b // bn,),
      in_specs=[
          pl.BlockSpec((bn, D), lambda i: (i, 0)),
          pl.BlockSpec((bn, D), lambda i: (i, 0)),
          pl.BlockSpec((D, D), lambda i: (0, 0)),
      ],
      out_specs=pl.BlockSpec((bn, D), lambda i: (i, 0)),
      out_shape=jax.ShapeDtypeStruct((b, D), jnp.float32),
  )(nbr_sum, rel_out, weight)


def _nbr_sum_half(neighbors, node_features):
  b = neighbors.shape[0]
  total_chunks = (b * S) // (NG * GSUB)
  assert total_chunks % 2 == 0
  npairs = total_chunks // 2
  base = npairs // NW
  w_hi = npairs - base * NW
  cnt_hi, cnt_lo = 2 * (base + 1), 2 * base
  if w_hi == 0:
    w_hi, cnt_hi = NW, cnt_lo
  idx = neighbors.reshape((b * S) // GSUB, GSUB)
  return _make_sc_kernel(b, cnt_hi, cnt_lo, w_hi)(node_features, idx)


def kernel(nodes, sampled_neighbors, sampled_relations, node_features, weight,
           relation_table):
  del nodes
  b, s = sampled_neighbors.shape
  assert s == S and node_features.shape[1] == D
  assert (b * S) % (2 * NG * GSUB) == 0 and b % 1000 == 0

  nbr_sum = _nbr_sum_half(sampled_neighbors, node_features)

  table_pad = jnp.pad(relation_table,
                      ((0, D - relation_table.shape[0]), (0, 0)))
  rel_out = _tc_rel(sampled_relations, table_pad)
  return _tc_final(nbr_sum, rel_out, weight)
